# baseline (speedup 1.0000x reference)
import functools

import jax
import jax.numpy as jnp
from jax import lax
from jax.experimental import pallas as pl
from jax.experimental.pallas import tpu as pltpu
from jax.experimental.pallas import tpu_sc as plsc

N = 10000
E = 320000
NC, NS = 2, 16
NW = NC * NS
CH = 80
NBUF = 4
NCHUNK = 125
NGROUP = 30
RPT = 632
NPAD = RPT * NS
ZR = 32
assert NW * NCHUNK * CH == E


def _mesh():
    return plsc.VectorSubcoreMesh(core_axis_name="c", subcore_axis_name="s")


_SC_PARAMS = pltpu.CompilerParams(use_tc_tiling_on_sc=False)


EPW = NCHUNK * CH


def _zero_acc(zbuf, acc, s):
    nfull = RPT // ZR
    for k in range(nfull + 1):
        cnt = ZR if k < nfull else RPT - nfull * ZR
        if cnt:
            pltpu.sync_copy(zbuf.at[pl.ds(0, cnt)],
                            acc.at[pl.ds(s * RPT + k * ZR, cnt)])


def _make_sc_scatter(D):
    @functools.partial(
        pl.kernel,
        out_type=jax.ShapeDtypeStruct((NC, NPAD, D), jnp.float32),
        mesh=_mesh(),
        compiler_params=_SC_PARAMS,
        scratch_types=[
            [pltpu.VMEM((CH,), jnp.int32) for _ in range(2 * NBUF)],
            [pltpu.VMEM((CH,), jnp.int32) for _ in range(NBUF)],
            [pltpu.VMEM((CH, D), jnp.float32) for _ in range(NBUF)],
            pltpu.VMEM((ZR, D), jnp.float32),
            pltpu.VMEM_SHARED((NPAD, D), jnp.float32),
            [pltpu.SemaphoreType.DMA for _ in range(2 * NBUF)],
            [pltpu.SemaphoreType.DMA for _ in range(NBUF)],
        ],
    )
    def k(edge_hbm, table_hbm, zeros_hbm, out_hbm,
          sslot, dslot, rows, zbuf, acc, isem, gsem):
        c = lax.axis_index("c")
        s = lax.axis_index("s")
        wid = s * NC + c
        base = wid * EPW

        def idx_issue(j, sb):
            pltpu.async_copy(edge_hbm.at[0, pl.ds(base + j * CH, CH)],
                             sslot[sb], isem[sb])

        def gat_issue(j, b, sb):
            pltpu.make_async_copy(
                edge_hbm.at[0, pl.ds(base + j * CH, CH)], sslot[sb],
                isem[sb]).wait()
            pltpu.async_copy(table_hbm.at[sslot[sb]], rows[b], gsem[b])
            pltpu.async_copy(edge_hbm.at[1, pl.ds(base + j * CH, CH)],
                             dslot[b], gsem[b])

        def step(j, b, sb, sbn, gat, idx):
            pltpu.make_async_copy(
                table_hbm.at[sslot[sb]], rows[b], gsem[b]).wait()
            pltpu.make_async_copy(
                edge_hbm.at[1, pl.ds(base + j * CH, CH)], dslot[b],
                gsem[b]).wait()
            pltpu.sync_copy(rows[b], acc.at[dslot[b]], add=True)
            if gat:
                gat_issue(j + NBUF, b, sbn)
            if idx:
                idx_issue(j + 2 * NBUF, sb)

        for j in range(NBUF):
            idx_issue(j, j)
        for j in range(NBUF):
            gat_issue(j, j, j)
            idx_issue(j + NBUF, j + NBUF)
        pltpu.sync_copy(zeros_hbm, zbuf)
        _zero_acc(zbuf, acc, s)
        plsc.subcore_barrier()

        def outer(g, carry):
            for t in range(2 * NBUF):
                j = g * 2 * NBUF + t
                step(j, t % NBUF, t, (t + NBUF) % (2 * NBUF), True, True)
            return carry

        ngr = (NCHUNK - 2 * NBUF - 1) // (2 * NBUF)
        lax.fori_loop(0, ngr, outer, 0)
        for j in range(ngr * 2 * NBUF, NCHUNK):
            step(j, j % NBUF, j % (2 * NBUF), (j + NBUF) % (2 * NBUF),
                 j + NBUF < NCHUNK, j + 2 * NBUF < NCHUNK)

        plsc.subcore_barrier()
        pltpu.sync_copy(acc.at[pl.ds(s * RPT, RPT)],
                        out_hbm.at[c, pl.ds(s * RPT, RPT)])

    return k


def _make_sc_degree():
    D = 8

    @functools.partial(
        pl.kernel,
        out_type=jax.ShapeDtypeStruct((NC, NPAD, D), jnp.float32),
        mesh=_mesh(),
        compiler_params=_SC_PARAMS,
        scratch_types=[
            [pltpu.VMEM((CH,), jnp.int32) for _ in range(8)],
            pltpu.VMEM((CH, D), jnp.float32),
            pltpu.VMEM((ZR, D), jnp.float32),
            pltpu.VMEM_SHARED((NPAD, D), jnp.float32),
            [pltpu.SemaphoreType.DMA for _ in range(8)],
            [pltpu.SemaphoreType.DMA for _ in range(4)],
        ],
    )
    def k(edge_hbm, ones_hbm, zeros_hbm, out_hbm, dslot, ones_v, zbuf,
          acc, isem, ssem):
        c = lax.axis_index("c")
        s = lax.axis_index("s")
        wid = s * NC + c
        base = wid * EPW

        def idx_issue(j, sb):
            pltpu.async_copy(edge_hbm.at[1, pl.ds(base + j * CH, CH)],
                             dslot[sb], isem[sb])

        def step(j, b, sb, sb4, drain, idx):
            pltpu.make_async_copy(
                edge_hbm.at[1, pl.ds(base + j * CH, CH)], dslot[sb],
                isem[sb]).wait()
            if drain:
                pltpu.make_async_copy(ones_v, acc.at[dslot[sb4]],
                                      ssem[b]).wait()
            pltpu.async_copy(ones_v, acc.at[dslot[sb]], ssem[b], add=True)
            if idx:
                idx_issue(j + 4, sb4)

        for j in range(4):
            idx_issue(j, j)
        pltpu.sync_copy(ones_hbm, ones_v)
        pltpu.sync_copy(zeros_hbm, zbuf)
        _zero_acc(zbuf, acc, s)
        plsc.subcore_barrier()

        for k0 in range(8):
            step(k0, k0 % 4, k0 % 8, (k0 + 4) % 8, k0 >= 4, True)

        def body(g, carry):
            for t in range(8):
                j = g * 8 + t
                step(j, t % 4, t % 8, (t + 4) % 8, True, True)
            return carry

        lax.fori_loop(1, 15, body, 0)
        for k0 in range(120, NCHUNK):
            step(k0, k0 % 4, k0 % 8, (k0 + 4) % 8, True, k0 + 4 < NCHUNK)
        for b in range(4):
            pltpu.make_async_copy(ones_v, acc.at[dslot[b]], ssem[b]).wait()
        plsc.subcore_barrier()
        pltpu.sync_copy(acc.at[pl.ds(s * RPT, RPT)],
                        out_hbm.at[c, pl.ds(s * RPT, RPT)])

    return k


_sc_deg = _make_sc_degree()
_sc_scat128 = _make_sc_scatter(128)
_sc_scat64 = _make_sc_scatter(64)


def _tc_mm_body(x_ref, w1_ref, u_ref):
    u_ref[...] = jnp.dot(x_ref[...], w1_ref[...],
                         preferred_element_type=jnp.float32)


def _tc_mm(x, W1):
    return pl.pallas_call(
        _tc_mm_body,
        out_shape=jax.ShapeDtypeStruct((N, 128), jnp.float32),
    )(x, W1)


def _tc_prep_body(degp_ref, u_ref, p1_ref, dinv_ref):
    deg = degp_ref[0, :N, 0:1] + degp_ref[1, :N, 0:1] + 1.0
    dinv = lax.rsqrt(deg)
    p1_ref[...] = u_ref[...] * dinv
    dinv_ref[...] = dinv


def _tc_prep(degp, u):
    return pl.pallas_call(
        _tc_prep_body,
        out_shape=(
            jax.ShapeDtypeStruct((N, 128), jnp.float32),
            jax.ShapeDtypeStruct((N, 1), jnp.float32),
        ),
    )(degp, u)


def _tc_mid_body(s1_ref, p1_ref, dinv_ref, b1_ref, bnw_ref, bnb_ref, w2_ref,
                 p2_ref):
    dinv = dinv_ref[...]
    hpre = (s1_ref[0, :N, :] + s1_ref[1, :N, :] + p1_ref[...]) * dinv \
        + b1_ref[...]
    mean = jnp.mean(hpre, axis=0, keepdims=True)
    var = jnp.mean((hpre - mean) * (hpre - mean), axis=0, keepdims=True)
    hn = (hpre - mean) * lax.rsqrt(var + 1e-5) * bnw_ref[...] + bnb_ref[...]
    h = jnp.maximum(hn, 0.0)
    p2_ref[...] = jnp.dot(h, w2_ref[...],
                          preferred_element_type=jnp.float32) * dinv


def _tc_mid(S1, p1, dinv, b1, bnw, bnb, W2):
    return pl.pallas_call(
        _tc_mid_body,
        out_shape=jax.ShapeDtypeStruct((N, 64), jnp.float32),
    )(S1, p1, dinv, b1, bnw, bnb, W2)


def _tc_final_body(s2_ref, p2_ref, dinv_ref, b2_ref, z_ref):
    z_ref[...] = (s2_ref[0, :N, :] + s2_ref[1, :N, :] + p2_ref[...]) \
        * dinv_ref[...] + b2_ref[...]


def _tc_final(S2, p2, dinv, b2):
    return pl.pallas_call(
        _tc_final_body,
        out_shape=jax.ShapeDtypeStruct((N, 64), jnp.float32),
    )(S2, p2, dinv, b2)


@jax.jit
def kernel(x, edge_index, W1, b1, bn_weight, bn_bias, W2, b2):
    ones16 = jnp.ones((CH, 8), jnp.float32)
    zeros16 = jnp.zeros((ZR, 8), jnp.float32)
    zeros128 = jnp.zeros((ZR, 128), jnp.float32)
    zeros64 = jnp.zeros((ZR, 64), jnp.float32)

    degp = _sc_deg(edge_index, ones16, zeros16)
    u = _tc_mm(x, W1)
    p1, dinv = _tc_prep(degp, u)
    S1 = _sc_scat128(edge_index, p1, zeros128)
    p2 = _tc_mid(S1, p1, dinv, b1.reshape(1, -1), bn_weight.reshape(1, -1),
                 bn_bias.reshape(1, -1), W2)
    S2 = _sc_scat64(edge_index, p2, zeros64)
    z = _tc_final(S2, p2, dinv, b2.reshape(1, -1))
    return z


# speedup vs baseline: 1.0065x; 1.0065x over previous
import functools

import jax
import jax.numpy as jnp
from jax import lax
from jax.experimental import pallas as pl
from jax.experimental.pallas import tpu as pltpu
from jax.experimental.pallas import tpu_sc as plsc

N = 10000
E = 320000
NC, NS = 2, 16
NW = NC * NS
CH = 80
NBUF = 4
NCHUNK = 125
NGROUP = 30
RPT = 632
NPAD = RPT * NS
ZR = 32
assert NW * NCHUNK * CH == E


def _mesh():
    return plsc.VectorSubcoreMesh(core_axis_name="c", subcore_axis_name="s")


_SC_PARAMS = pltpu.CompilerParams(use_tc_tiling_on_sc=False)


EPW = NCHUNK * CH


def _zero_acc(zbuf, acc, s):
    nfull = RPT // ZR
    for k in range(nfull + 1):
        cnt = ZR if k < nfull else RPT - nfull * ZR
        if cnt:
            pltpu.sync_copy(zbuf.at[pl.ds(0, cnt)],
                            acc.at[pl.ds(s * RPT + k * ZR, cnt)])


def _make_sc_scatter(D):
    @functools.partial(
        pl.kernel,
        out_type=jax.ShapeDtypeStruct((NC, NPAD, D), jnp.float32),
        mesh=_mesh(),
        compiler_params=_SC_PARAMS,
        scratch_types=[
            [pltpu.VMEM((CH,), jnp.int32) for _ in range(2 * NBUF)],
            [pltpu.VMEM((CH,), jnp.int32) for _ in range(NBUF)],
            [pltpu.VMEM((CH, D), jnp.float32) for _ in range(NBUF)],
            pltpu.VMEM((ZR, D), jnp.float32),
            pltpu.VMEM_SHARED((NPAD, D), jnp.float32),
            [pltpu.SemaphoreType.DMA for _ in range(2 * NBUF)],
            [pltpu.SemaphoreType.DMA for _ in range(NBUF)],
        ],
    )
    def k(edge_hbm, table_hbm, zeros_hbm, out_hbm,
          sslot, dslot, rows, zbuf, acc, isem, gsem):
        c = lax.axis_index("c")
        s = lax.axis_index("s")
        wid = s * NC + c
        base = wid * EPW

        def idx_issue(j, sb):
            pltpu.async_copy(edge_hbm.at[0, pl.ds(base + j * CH, CH)],
                             sslot[sb], isem[sb])

        def gat_issue(j, b, sb):
            pltpu.make_async_copy(
                edge_hbm.at[0, pl.ds(base + j * CH, CH)], sslot[sb],
                isem[sb]).wait()
            pltpu.async_copy(table_hbm.at[sslot[sb]], rows[b], gsem[b])
            pltpu.async_copy(edge_hbm.at[1, pl.ds(base + j * CH, CH)],
                             dslot[b], gsem[b])

        def step(j, b, sb, sbn, gat, idx):
            pltpu.make_async_copy(
                table_hbm.at[sslot[sb]], rows[b], gsem[b]).wait()
            pltpu.make_async_copy(
                edge_hbm.at[1, pl.ds(base + j * CH, CH)], dslot[b],
                gsem[b]).wait()
            pltpu.sync_copy(rows[b], acc.at[dslot[b]], add=True)
            if gat:
                gat_issue(j + NBUF, b, sbn)
            if idx:
                idx_issue(j + 2 * NBUF, sb)

        for j in range(NBUF):
            idx_issue(j, j)
        for j in range(NBUF):
            gat_issue(j, j, j)
            idx_issue(j + NBUF, j + NBUF)
        pltpu.sync_copy(zeros_hbm, zbuf)
        _zero_acc(zbuf, acc, s)
        plsc.subcore_barrier()

        def outer(g, carry):
            for t in range(2 * NBUF):
                j = g * 2 * NBUF + t
                step(j, t % NBUF, t, (t + NBUF) % (2 * NBUF), True, True)
            return carry

        ngr = (NCHUNK - 2 * NBUF - 1) // (2 * NBUF)
        lax.fori_loop(0, ngr, outer, 0)
        for j in range(ngr * 2 * NBUF, NCHUNK):
            step(j, j % NBUF, j % (2 * NBUF), (j + NBUF) % (2 * NBUF),
                 j + NBUF < NCHUNK, j + 2 * NBUF < NCHUNK)

        plsc.subcore_barrier()
        pltpu.sync_copy(acc.at[pl.ds(s * RPT, RPT)],
                        out_hbm.at[c, pl.ds(s * RPT, RPT)])

    return k


def _make_sc_degree():
    D = 16

    @functools.partial(
        pl.kernel,
        out_type=jax.ShapeDtypeStruct((NC, NPAD, D), jnp.float32),
        mesh=_mesh(),
        compiler_params=_SC_PARAMS,
        scratch_types=[
            [pltpu.VMEM((CH,), jnp.int32) for _ in range(8)],
            pltpu.VMEM((CH, D), jnp.float32),
            pltpu.VMEM((ZR, D), jnp.float32),
            pltpu.VMEM_SHARED((NPAD, D), jnp.float32),
            [pltpu.SemaphoreType.DMA for _ in range(8)],
            [pltpu.SemaphoreType.DMA for _ in range(4)],
        ],
    )
    def k(edge_hbm, ones_hbm, zeros_hbm, out_hbm, dslot, ones_v, zbuf,
          acc, isem, ssem):
        c = lax.axis_index("c")
        s = lax.axis_index("s")
        wid = s * NC + c
        base = wid * EPW

        def idx_issue(j, sb):
            pltpu.async_copy(edge_hbm.at[1, pl.ds(base + j * CH, CH)],
                             dslot[sb], isem[sb])

        def step(j, b, sb, sb4, drain, idx):
            pltpu.make_async_copy(
                edge_hbm.at[1, pl.ds(base + j * CH, CH)], dslot[sb],
                isem[sb]).wait()
            if drain:
                pltpu.make_async_copy(ones_v, acc.at[dslot[sb4]],
                                      ssem[b]).wait()
            pltpu.async_copy(ones_v, acc.at[dslot[sb]], ssem[b], add=True)
            if idx:
                idx_issue(j + 4, sb4)

        for j in range(4):
            idx_issue(j, j)
        pltpu.sync_copy(ones_hbm, ones_v)
        pltpu.sync_copy(zeros_hbm, zbuf)
        _zero_acc(zbuf, acc, s)
        plsc.subcore_barrier()

        for k0 in range(8):
            step(k0, k0 % 4, k0 % 8, (k0 + 4) % 8, k0 >= 4, True)

        def body(g, carry):
            for t in range(8):
                j = g * 8 + t
                step(j, t % 4, t % 8, (t + 4) % 8, True, True)
            return carry

        lax.fori_loop(1, 15, body, 0)
        for k0 in range(120, NCHUNK):
            step(k0, k0 % 4, k0 % 8, (k0 + 4) % 8, True, k0 + 4 < NCHUNK)
        for b in range(4):
            pltpu.make_async_copy(ones_v, acc.at[dslot[b]], ssem[b]).wait()
        plsc.subcore_barrier()
        pltpu.sync_copy(acc.at[pl.ds(s * RPT, RPT)],
                        out_hbm.at[c, pl.ds(s * RPT, RPT)])

    return k


_sc_deg = _make_sc_degree()
_sc_scat128 = _make_sc_scatter(128)
_sc_scat64 = _make_sc_scatter(64)


def _tc_prep_body(degp_ref, x_ref, w1_ref, p1_ref, dinv_ref):
    deg = degp_ref[0, :N, 0:1] + degp_ref[1, :N, 0:1] + 1.0
    dinv = lax.rsqrt(deg)
    u = jnp.dot(x_ref[...], w1_ref[...], preferred_element_type=jnp.float32)
    p1_ref[...] = u * dinv
    dinv_ref[...] = dinv


def _tc_prep(degp, x, W1):
    return pl.pallas_call(
        _tc_prep_body,
        out_shape=(
            jax.ShapeDtypeStruct((N, 128), jnp.float32),
            jax.ShapeDtypeStruct((N, 1), jnp.float32),
        ),
    )(degp, x, W1)


def _tc_mid_body(s1_ref, p1_ref, dinv_ref, b1_ref, bnw_ref, bnb_ref, w2_ref,
                 p2_ref):
    dinv = dinv_ref[...]
    hpre = (s1_ref[0, :N, :] + s1_ref[1, :N, :] + p1_ref[...]) * dinv \
        + b1_ref[...]
    mean = jnp.mean(hpre, axis=0, keepdims=True)
    var = jnp.mean((hpre - mean) * (hpre - mean), axis=0, keepdims=True)
    hn = (hpre - mean) * lax.rsqrt(var + 1e-5) * bnw_ref[...] + bnb_ref[...]
    h = jnp.maximum(hn, 0.0)
    p2_ref[...] = jnp.dot(h, w2_ref[...],
                          preferred_element_type=jnp.float32) * dinv


def _tc_mid(S1, p1, dinv, b1, bnw, bnb, W2):
    return pl.pallas_call(
        _tc_mid_body,
        out_shape=jax.ShapeDtypeStruct((N, 64), jnp.float32),
    )(S1, p1, dinv, b1, bnw, bnb, W2)


def _tc_final_body(s2_ref, p2_ref, dinv_ref, b2_ref, z_ref):
    z_ref[...] = (s2_ref[0, :N, :] + s2_ref[1, :N, :] + p2_ref[...]) \
        * dinv_ref[...] + b2_ref[...]


def _tc_final(S2, p2, dinv, b2):
    return pl.pallas_call(
        _tc_final_body,
        out_shape=jax.ShapeDtypeStruct((N, 64), jnp.float32),
    )(S2, p2, dinv, b2)


@jax.jit
def kernel(x, edge_index, W1, b1, bn_weight, bn_bias, W2, b2):
    ones16 = jnp.ones((CH, 16), jnp.float32)
    zeros16 = jnp.zeros((ZR, 16), jnp.float32)
    zeros128 = jnp.zeros((ZR, 128), jnp.float32)
    zeros64 = jnp.zeros((ZR, 64), jnp.float32)

    degp = _sc_deg(edge_index, ones16, zeros16)
    p1, dinv = _tc_prep(degp, x, W1)
    S1 = _sc_scat128(edge_index, p1, zeros128)
    p2 = _tc_mid(S1, p1, dinv, b1.reshape(1, -1), bn_weight.reshape(1, -1),
                 bn_bias.reshape(1, -1), W2)
    S2 = _sc_scat64(edge_index, p2, zeros64)
    z = _tc_final(S2, p2, dinv, b2.reshape(1, -1))
    return z
